# Initial kernel scaffold; baseline (speedup 1.0000x reference)
import jax
import jax.numpy as jnp
from jax.experimental import pallas as pl


def kernel(hour, session_emb, hour_emb, W, b):
    raise NotImplementedError("write your pallas kernel here")


# speedup vs baseline: 3.0587x; 3.0587x over previous
import functools

import jax
import jax.numpy as jnp
from jax import lax
from jax.experimental import pallas as pl
from jax.experimental.pallas import tpu as pltpu
from jax.experimental.pallas import tpu_sc as plsc

D3 = 21
DM = 64
NHOUR = 24
BATCH, SEQ = 16384, 200
ROWS_TOTAL = BATCH * SEQ
LANE = 128
NBLK = ROWS_TOTAL // LANE
NW = 32
BLK_PER_W = NBLK // NW
CH = 8
NIT = BLK_PER_W // CH


def _table_body(ses_ref, hr_ref, w_ref, b_ref, out_ref):
    h = lax.broadcasted_iota(jnp.int32, (NHOUR, D3), 0)
    r0 = jnp.broadcast_to(ses_ref[0:1, :], (NHOUR, D3))
    r1 = jnp.broadcast_to(ses_ref[1:2, :], (NHOUR, D3))
    r2 = jnp.broadcast_to(ses_ref[2:3, :], (NHOUR, D3))
    ses = jnp.where(h < 8, r0, jnp.where(h < 16, r1, r2))
    ws = w_ref[:, :D3]
    wh = w_ref[:, D3:]
    t = lax.dot_general(ses, ws, (((1,), (1,)), ((), ())),
                        preferred_element_type=jnp.float32)
    t = t + lax.dot_general(hr_ref[...], wh, (((1,), (1,)), ((), ())),
                            preferred_element_type=jnp.float32)
    out_ref[...] = t + b_ref[...]


_table_tc = pl.pallas_call(
    _table_body,
    out_shape=jax.ShapeDtypeStruct((NHOUR, DM), jnp.float32),
)


_mesh = plsc.VectorSubcoreMesh(core_axis_name="c", subcore_axis_name="s")


@functools.partial(
    pl.kernel,
    mesh=_mesh,
    out_type=jax.ShapeDtypeStruct((NBLK, LANE, DM), jnp.float32),
    scratch_types=[
        pltpu.VMEM((CH, LANE), jnp.int32),
        pltpu.VMEM((CH, LANE, DM), jnp.float32),
        pltpu.SemaphoreType.DMA,
    ],
    compiler_params=pltpu.CompilerParams(use_tc_tiling_on_sc=False),
)
def _gather_sc(table_hbm, hour_hbm, out_hbm, idx_v, rows_v, sem):
    wid = lax.axis_index("s") * 2 + lax.axis_index("c")
    base = wid * BLK_PER_W

    def body(i, carry):
        blk0 = base + i * CH
        pltpu.sync_copy(hour_hbm.at[pl.ds(blk0, CH)], idx_v)
        handles = [
            pltpu.async_copy(table_hbm.at[idx_v.at[j]], rows_v.at[j], sem)
            for j in range(CH)
        ]
        for hnd in handles:
            hnd.wait()
        pltpu.sync_copy(rows_v, out_hbm.at[pl.ds(blk0, CH)])
        return carry

    lax.fori_loop(0, NIT, body, 0)


def kernel(hour, session_emb, hour_emb, W, b):
    table = _table_tc(session_emb, hour_emb, W, b.reshape(1, DM))
    hour2d = hour.astype(jnp.int32).reshape(NBLK, LANE)
    out = _gather_sc(table, hour2d)
    return out.reshape(BATCH, SEQ, DM)


# speedup vs baseline: 3.0645x; 1.0019x over previous
import functools

import jax
import jax.numpy as jnp
from jax import lax
from jax.experimental import pallas as pl
from jax.experimental.pallas import tpu as pltpu
from jax.experimental.pallas import tpu_sc as plsc

D3 = 21
DM = 64
NHOUR = 24
BATCH, SEQ = 16384, 200
ROWS_TOTAL = BATCH * SEQ
LANE = 128
NBLK = ROWS_TOTAL // LANE
NW = 32
BLK_PER_W = NBLK // NW
CH = 4
NCHUNK = BLK_PER_W // CH
KMID = (NCHUNK - 8) // 4
assert NCHUNK == 8 + 4 * KMID


def _table_body(ses_ref, hr_ref, w_ref, b_ref, out_ref):
    h = lax.broadcasted_iota(jnp.int32, (NHOUR, D3), 0)
    r0 = jnp.broadcast_to(ses_ref[0:1, :], (NHOUR, D3))
    r1 = jnp.broadcast_to(ses_ref[1:2, :], (NHOUR, D3))
    r2 = jnp.broadcast_to(ses_ref[2:3, :], (NHOUR, D3))
    ses = jnp.where(h < 8, r0, jnp.where(h < 16, r1, r2))
    ws = w_ref[:, :D3]
    wh = w_ref[:, D3:]
    t = lax.dot_general(ses, ws, (((1,), (1,)), ((), ())),
                        preferred_element_type=jnp.float32)
    t = t + lax.dot_general(hr_ref[...], wh, (((1,), (1,)), ((), ())),
                            preferred_element_type=jnp.float32)
    out_ref[...] = t + b_ref[...]


_table_tc = pl.pallas_call(
    _table_body,
    out_shape=jax.ShapeDtypeStruct((NHOUR, DM), jnp.float32),
)


_mesh = plsc.VectorSubcoreMesh(core_axis_name="c", subcore_axis_name="s")


@functools.partial(
    pl.kernel,
    mesh=_mesh,
    out_type=jax.ShapeDtypeStruct((NBLK, LANE, DM), jnp.float32),
    scratch_types=[
        pltpu.VMEM((CH, LANE), jnp.int32),
        pltpu.VMEM((CH, LANE), jnp.int32),
        pltpu.VMEM((CH, LANE), jnp.int32),
        pltpu.VMEM((CH, LANE), jnp.int32),
        pltpu.VMEM((CH, LANE, DM), jnp.float32),
        pltpu.VMEM((CH, LANE, DM), jnp.float32),
        pltpu.SemaphoreType.DMA,
        pltpu.SemaphoreType.DMA,
        pltpu.SemaphoreType.DMA,
        pltpu.SemaphoreType.DMA,
        pltpu.SemaphoreType.DMA,
        pltpu.SemaphoreType.DMA,
        pltpu.SemaphoreType.DMA,
        pltpu.SemaphoreType.DMA,
    ],
    compiler_params=pltpu.CompilerParams(use_tc_tiling_on_sc=False),
)
def _gather_sc(table_hbm, hour_hbm, out_hbm,
               idx0, idx1, idx2, idx3, rows0, rows1,
               si0, si1, si2, si3, sg0, sg1, sw0, sw1):
    idx = [idx0, idx1, idx2, idx3]
    rows = [rows0, rows1]
    si = [si0, si1, si2, si3]
    sg = [sg0, sg1]
    sw = [sw0, sw1]

    wid = lax.axis_index("s") * 2 + lax.axis_index("c")
    base = wid * BLK_PER_W

    def blk0(g):
        return base + g * CH

    def fire_idx(g, q):
        pltpu.async_copy(hour_hbm.at[pl.ds(blk0(g), CH)], idx[q], si[q])

    def wait_idx(q):
        pltpu.make_async_copy(hour_hbm.at[pl.ds(base, CH)], idx[q], si[q]).wait()

    def fire_gather(g, p, q):
        for j in range(CH):
            pltpu.async_copy(table_hbm.at[idx[q].at[j]], rows[p].at[j], sg[p])

    def wait_gather(p, q):
        for j in range(CH):
            pltpu.make_async_copy(table_hbm.at[idx[q].at[j]], rows[p].at[j],
                                  sg[p]).wait()

    def fire_wb(g, p):
        pltpu.async_copy(rows[p], out_hbm.at[pl.ds(blk0(g), CH)], sw[p])

    def wait_wb(p):
        pltpu.make_async_copy(rows[p], out_hbm.at[pl.ds(base, CH)], sw[p]).wait()

    def step(g, u):
        p, q = u % 2, u % 4
        wait_idx(q)
        wait_wb(p)
        fire_gather(g, p, q)
        wait_gather(1 - p, (q - 1) % 4)
        fire_wb(g - 1, 1 - p)
        fire_idx(g + 3, (q + 3) % 4)

    for g in range(4):
        fire_idx(g, g)
    wait_idx(0)
    fire_gather(0, 0, 0)
    wait_idx(1)
    fire_gather(1, 1, 1)
    wait_gather(0, 0)
    fire_wb(0, 0)
    fire_idx(4, 0)
    wait_idx(2)
    wait_wb(0)
    fire_gather(2, 0, 2)
    wait_gather(1, 1)
    fire_wb(1, 1)
    fire_idx(5, 1)
    wait_idx(3)
    wait_wb(1)
    fire_gather(3, 1, 3)
    wait_gather(0, 2)
    fire_wb(2, 0)
    fire_idx(6, 2)

    def body(k, carry):
        for u in range(4):
            step(4 * k + u, u)
        return carry

    lax.fori_loop(1, KMID + 1, body, 0)

    for g in range(NCHUNK - 4, NCHUNK):
        u = g % 4
        p, q = u % 2, u % 4
        wait_idx(q)
        wait_wb(p)
        fire_gather(g, p, q)
        wait_gather(1 - p, (q - 1) % 4)
        fire_wb(g - 1, 1 - p)
        if g + 3 < NCHUNK:
            fire_idx(g + 3, (q + 3) % 4)
    wait_gather((NCHUNK - 1) % 2, (NCHUNK - 1) % 4)
    fire_wb(NCHUNK - 1, (NCHUNK - 1) % 2)
    wait_wb(0)
    wait_wb(1)


def kernel(hour, session_emb, hour_emb, W, b):
    table = _table_tc(session_emb, hour_emb, W, b.reshape(1, DM))
    hour2d = hour.astype(jnp.int32).reshape(NBLK, LANE)
    out = _gather_sc(table, hour2d)
    return out.reshape(BATCH, SEQ, DM)
